# baseline (speedup 1.0000x reference)
import math

import jax
import jax.numpy as jnp
import numpy as np
from jax import lax
from jax.experimental import pallas as pl
from jax.experimental.pallas import tpu as pltpu
from jax.experimental.pallas import tpu_sc as plsc

DICT_SIZE = 1000000
D = 64
L_SEQ = 200
B = 1024
NW = 32
ROWS_PER_W = B // NW
LCH = 4
CHUNK = LCH * ROWS_PER_W
NCHUNK = L_SEQ // LCH
LANES = 16
NVREG_ROW = D // LANES
SCALE = math.sqrt(D)
NIDX = ROWS_PER_W * L_SEQ


def _positional_encoding_np(seq_len, d_model):
    pos = np.arange(seq_len, dtype=np.float32)[:, None]
    div = np.exp(
        np.arange(0, d_model, 2, dtype=np.float32)
        * (-math.log(10000.0) / d_model)
    )
    pe = np.zeros((seq_len, d_model), dtype=np.float32)
    pe[:, 0::2] = np.sin(pos * div)
    pe[:, 1::2] = np.cos(pos * div)
    return pe


_PE = _positional_encoding_np(L_SEQ, D)


def _sc_body(x_hbm, pe_hbm, emb_hbm, out_hbm,
             idx_v, idx_t, pe_v, bufs, obuf, gsem, wsem):
    c = lax.axis_index("c")
    s = lax.axis_index("s")
    w = s * 2 + c
    row0 = w * ROWS_PER_W
    btw = w // 4
    bc0 = (w % 4) * ROWS_PER_W

    pltpu.sync_copy(x_hbm.at[pl.ds(row0, ROWS_PER_W)], idx_v)
    pltpu.sync_copy(pe_hbm, pe_v)

    iota16 = lax.iota(jnp.int32, LANES)
    ones16 = jnp.ones((LANES,), jnp.int32)
    dtv = [
        jax.lax.shift_right_logical(iota16 + k * LANES, 3)
        for k in range(NVREG_ROW)
    ]
    drv = [(iota16 + k * LANES) & 7 for k in range(NVREG_ROW)]

    @pl.loop(0, ROWS_PER_W)
    def _(bb):
        for li in range(13):
            l0 = min(li * LANES, L_SEQ - LANES)
            v = idx_v[bb, pl.ds(l0, LANES)]
            plsc.store_scatter(
                idx_t, [(iota16 + l0) * ROWS_PER_W + bb], v
            )

    def gather_chunk(q, bN):
        pltpu.async_copy(
            emb_hbm.at[idx_t.at[pl.ds(q * CHUNK, CHUNK)]],
            bufs.at[bN],
            gsem.at[bN],
        )

    def wait_gather(bN):
        pltpu.make_async_copy(
            emb_hbm.at[pl.ds(0, CHUNK)], bufs.at[bN], gsem.at[bN]
        ).wait()

    def compute_chunk(q, bN):
        bn_splat = ones16 * bN

        @pl.loop(0, CHUNK)
        def _(t):
            ll = t >> 5
            bb = t & 31
            l = q * LCH + ll
            ll_splat = ones16 * ll
            bb_splat = ones16 * bb
            for k in range(NVREG_ROW):
                sl = pl.ds(k * LANES, LANES)
                val = bufs[bN, t, sl] * SCALE + pe_v[l, sl]
                plsc.store_scatter(
                    obuf, [bn_splat, ll_splat, dtv[k], drv[k], bb_splat], val
                )

    def wb_chunk(q, bN):
        pltpu.async_copy(
            obuf.at[bN],
            out_hbm.at[
                pl.ds(q * LCH, LCH), :, btw, :, pl.ds(bc0, ROWS_PER_W)
            ],
            wsem.at[bN],
        )

    def wait_wb(bN):
        pltpu.make_async_copy(
            obuf.at[bN],
            out_hbm.at[pl.ds(0, LCH), :, 0, :, pl.ds(0, ROWS_PER_W)],
            wsem.at[bN],
        ).wait()

    gather_chunk(0, 0)
    gather_chunk(1, 1)

    @pl.loop(0, NCHUNK, step=2)
    def _(qbase):
        for cc in range(2):
            q = qbase + cc
            wait_gather(cc)

            @pl.when(qbase > 0)
            def _():
                wait_wb(cc)

            compute_chunk(q, cc)
            wb_chunk(q, cc)

            @pl.when(q + 2 < NCHUNK)
            def _():
                gather_chunk(q + 2, cc)

    wait_wb(0)
    wait_wb(1)


@jax.jit
def _pre_layer_sc(x, pe, emb_weight):
    mesh = plsc.VectorSubcoreMesh(core_axis_name="c", subcore_axis_name="s")
    k = pl.kernel(
        _sc_body,
        out_type=jax.ShapeDtypeStruct((L_SEQ, 8, 8, 8, 128), jnp.float32),
        mesh=mesh,
        scratch_types=[
            pltpu.VMEM((ROWS_PER_W, L_SEQ), jnp.int32),
            pltpu.VMEM((NIDX,), jnp.int32),
            pltpu.VMEM((L_SEQ, D), jnp.float32),
            pltpu.VMEM((2, CHUNK, D), jnp.float32),
            pltpu.VMEM((2, LCH, 8, 8, ROWS_PER_W), jnp.float32),
            pltpu.SemaphoreType.DMA((2,)),
            pltpu.SemaphoreType.DMA((2,)),
        ],
        compiler_params=pltpu.CompilerParams(
            use_tc_tiling_on_sc=False, needs_layout_passes=False
        ),
    )
    return k(x, pe, emb_weight)


def kernel(x, emb_weight):
    pe = jnp.asarray(_PE)
    out5 = _pre_layer_sc(x.astype(jnp.int32), pe, emb_weight)
    return out5.transpose(2, 4, 0, 1, 3).reshape(B, L_SEQ, D)


# speedup vs baseline: 1.1792x; 1.1792x over previous
import math

import jax
import jax.numpy as jnp
import numpy as np
from jax import lax
from jax.experimental import pallas as pl
from jax.experimental.pallas import tpu as pltpu
from jax.experimental.pallas import tpu_sc as plsc

DICT_SIZE = 1000000
D = 64
L_SEQ = 200
B = 1024
NW = 32
ROWS_PER_W = B // NW
CHUNK = 40
NCHUNK = L_SEQ // CHUNK
LANES = 16
NVREG_ROW = D // LANES
NBUF = 4
NROUND = ROWS_PER_W // NBUF
SCALE = math.sqrt(D)
ROW_F = L_SEQ * D
OROWS = ROW_F // 128


def _positional_encoding_np(seq_len, d_model):
    pos = np.arange(seq_len, dtype=np.float32)[:, None]
    div = np.exp(
        np.arange(0, d_model, 2, dtype=np.float32)
        * (-math.log(10000.0) / d_model)
    )
    pe = np.zeros((seq_len, d_model), dtype=np.float32)
    pe[:, 0::2] = np.sin(pos * div)
    pe[:, 1::2] = np.cos(pos * div)
    return pe


_PE = _positional_encoding_np(L_SEQ, D)


def _sc_body(x_hbm, pe_hbm, emb_hbm, out_hbm,
             idx_v, pe_v, bufs, obuf, gsem, wsem):
    c = lax.axis_index("c")
    s = lax.axis_index("s")
    w = s * 2 + c
    row0 = w * ROWS_PER_W

    pltpu.sync_copy(x_hbm.at[pl.ds(row0, ROWS_PER_W)], idx_v)
    pltpu.sync_copy(pe_hbm, pe_v)

    def gather_row(r, b):
        for ch in range(NCHUNK):
            pltpu.async_copy(
                emb_hbm.at[idx_v.at[r, pl.ds(ch * CHUNK, CHUNK)]],
                bufs.at[b, pl.ds(ch * CHUNK, CHUNK)],
                gsem.at[b],
            )

    def wait_gather(b):
        pltpu.make_async_copy(
            emb_hbm.at[pl.ds(0, L_SEQ)], bufs.at[b], gsem.at[b]
        ).wait()

    def wb_row(r, b):
        pltpu.async_copy(
            obuf.at[b],
            out_hbm.at[pl.ds((row0 + r) * OROWS, OROWS)],
            wsem.at[b],
        )

    def wait_wb(b):
        pltpu.make_async_copy(
            obuf.at[b], out_hbm.at[pl.ds(0, OROWS)], wsem.at[b]
        ).wait()

    def compute(b):
        @plsc.parallel_loop(0, L_SEQ, unroll=8)
        def _(j):
            for k in range(NVREG_ROW):
                sl = pl.ds(k * LANES, LANES)
                q = j * NVREG_ROW + k
                obuf[b, q >> 3, pl.ds((q & 7) * LANES, LANES)] = (
                    bufs[b, j, sl] * SCALE + pe_v[j, sl]
                )

    for b in range(NBUF):
        gather_row(b, b)

    @pl.loop(0, NROUND)
    def _(g):
        r0 = g * NBUF
        for b in range(NBUF):
            wait_gather(b)

            @pl.when(g > 0)
            def _():
                wait_wb(b)

            compute(b)
            wb_row(r0 + b, b)
        @pl.when(g < NROUND - 1)
        def _():
            for b in range(NBUF):
                gather_row(r0 + NBUF + b, b)

    for b in range(NBUF):
        wait_wb(b)


@jax.jit
def _pre_layer_sc(x, pe, emb_weight):
    mesh = plsc.VectorSubcoreMesh(core_axis_name="c", subcore_axis_name="s")
    k = pl.kernel(
        _sc_body,
        out_type=jax.ShapeDtypeStruct((B * L_SEQ * D // 128, 128), jnp.float32),
        mesh=mesh,
        scratch_types=[
            pltpu.VMEM((ROWS_PER_W, L_SEQ), jnp.int32),
            pltpu.VMEM((L_SEQ, D), jnp.float32),
            pltpu.VMEM((NBUF, L_SEQ, D), jnp.float32),
            pltpu.VMEM((NBUF, OROWS, 128), jnp.float32),
            pltpu.SemaphoreType.DMA((NBUF,)),
            pltpu.SemaphoreType.DMA((NBUF,)),
        ],
        compiler_params=pltpu.CompilerParams(use_tc_tiling_on_sc=False),
    )
    return k(x, pe, emb_weight)


def kernel(x, emb_weight):
    pe = jnp.asarray(_PE)
    out = _pre_layer_sc(x.astype(jnp.int32), pe, emb_weight)
    return out.reshape(B, L_SEQ, D)
